# baseline (speedup 1.0000x reference)
import functools

import jax
import jax.numpy as jnp
from jax import lax
from jax.experimental import pallas as pl
from jax.experimental.pallas import tpu as pltpu
from jax.experimental.pallas import tpu_sc as plsc

_VOCAB = 1000000
_COL = 64
_BATCH = 4096
_SEQ = 200

_N = _BATCH * _SEQ
_NW = 32
_PER_W = _N // _NW
_CHUNK = 128
_NCHUNK = _PER_W // _CHUNK


def _gather_body(table_hbm, idx_hbm, out_hbm, idx_v, buf0, buf1, sem0, sem1):
    wid = lax.axis_index("s") * 2 + lax.axis_index("c")
    base = wid * _PER_W
    pltpu.sync_copy(idx_hbm.at[pl.ds(base, _PER_W)], idx_v)

    bufs = (buf0, buf1)
    sems = (sem0, sem1)

    def start(j, slot):
        return pltpu.async_copy(
            table_hbm.at[idx_v.at[pl.ds(j * _CHUNK, _CHUNK)]], bufs[slot], sems[slot]
        )

    start(0, 0)
    start(1, 1)

    def body(j, _):
        slot = lax.rem(j, 2)

        @pl.when(slot == 0)
        def _():
            pltpu.make_async_copy(
                table_hbm.at[idx_v.at[pl.ds(j * _CHUNK, _CHUNK)]], buf0, sem0
            ).wait()
            pltpu.sync_copy(buf0, out_hbm.at[pl.ds(base + j * _CHUNK, _CHUNK)])

            @pl.when(j + 2 < _NCHUNK)
            def _():
                start(j + 2, 0)

        @pl.when(slot == 1)
        def _():
            pltpu.make_async_copy(
                table_hbm.at[idx_v.at[pl.ds(j * _CHUNK, _CHUNK)]], buf1, sem1
            ).wait()
            pltpu.sync_copy(buf1, out_hbm.at[pl.ds(base + j * _CHUNK, _CHUNK)])

            @pl.when(j + 2 < _NCHUNK)
            def _():
                start(j + 2, 1)

        return 0

    lax.fori_loop(0, _NCHUNK, body, 0)


def kernel(x, embed_weight):
    idx = x.reshape(_N).astype(jnp.int32)
    mesh = plsc.VectorSubcoreMesh(core_axis_name="c", subcore_axis_name="s")

    gather = functools.partial(
        pl.kernel,
        mesh=mesh,
        out_type=jax.ShapeDtypeStruct((_N, _COL), jnp.float32),
        scratch_types=[
            pltpu.VMEM((_PER_W,), jnp.int32),
            pltpu.VMEM((_CHUNK, _COL), jnp.float32),
            pltpu.VMEM((_CHUNK, _COL), jnp.float32),
            pltpu.SemaphoreType.DMA,
            pltpu.SemaphoreType.DMA,
        ],
        compiler_params=pltpu.CompilerParams(use_tc_tiling_on_sc=False),
    )(_gather_body)

    out = gather(embed_weight, idx)
    return out.reshape(_BATCH, _SEQ, _COL)


# speedup vs baseline: 1.0202x; 1.0202x over previous
import functools

import jax
import jax.numpy as jnp
from jax import lax
from jax.experimental import pallas as pl
from jax.experimental.pallas import tpu as pltpu
from jax.experimental.pallas import tpu_sc as plsc

_VOCAB = 1000000
_COL = 64
_BATCH = 4096
_SEQ = 200

_N = _BATCH * _SEQ
_NW = 32
_PER_W = _N // _NW
_CHUNK = 128
_NCHUNK = _PER_W // _CHUNK
_NBUF = 4
_ITERS = _NCHUNK // _NBUF


def _gather_body(table_hbm, idx_hbm, out_hbm, idx_v, *scratch):
    bufs = scratch[:_NBUF]
    gsems = scratch[_NBUF:2 * _NBUF]
    wsems = scratch[2 * _NBUF:]

    wid = lax.axis_index("s") * 2 + lax.axis_index("c")
    base = wid * _PER_W
    pltpu.sync_copy(idx_hbm.at[pl.ds(base, _PER_W)], idx_v)

    def gstart(j, b):
        pltpu.async_copy(
            table_hbm.at[idx_v.at[pl.ds(j * _CHUNK, _CHUNK)]], bufs[b], gsems[b]
        )

    def gwait(j, b):
        pltpu.make_async_copy(
            table_hbm.at[idx_v.at[pl.ds(j * _CHUNK, _CHUNK)]], bufs[b], gsems[b]
        ).wait()

    def wstart(j, b):
        pltpu.async_copy(
            bufs[b], out_hbm.at[pl.ds(base + j * _CHUNK, _CHUNK)], wsems[b]
        )

    def wwait(j, b):
        pltpu.make_async_copy(
            bufs[b], out_hbm.at[pl.ds(base + j * _CHUNK, _CHUNK)], wsems[b]
        ).wait()

    for b in range(_NBUF):
        gstart(b, b)

    def body(it, _):
        for b in range(_NBUF):
            j = it * _NBUF + b
            gwait(j, b)
            wstart(j, b)

            @pl.when(it + 1 < _ITERS)
            def _():
                wwait(j, b)
                gstart(j + _NBUF, b)

        return 0

    lax.fori_loop(0, _ITERS, body, 0)

    for b in range(_NBUF):
        wwait((_ITERS - 1) * _NBUF + b, b)


def kernel(x, embed_weight):
    idx = x.reshape(_N).astype(jnp.int32)
    mesh = plsc.VectorSubcoreMesh(core_axis_name="c", subcore_axis_name="s")

    gather = functools.partial(
        pl.kernel,
        mesh=mesh,
        out_type=jax.ShapeDtypeStruct((_N, _COL), jnp.float32),
        scratch_types=(
            [pltpu.VMEM((_PER_W,), jnp.int32)]
            + [pltpu.VMEM((_CHUNK, _COL), jnp.float32) for _ in range(_NBUF)]
            + [pltpu.SemaphoreType.DMA for _ in range(2 * _NBUF)]
        ),
        compiler_params=pltpu.CompilerParams(use_tc_tiling_on_sc=False),
    )(_gather_body)

    out = gather(embed_weight, idx)
    return out.reshape(_BATCH, _SEQ, _COL)


# speedup vs baseline: 1.3575x; 1.3306x over previous
import functools

import jax
import jax.numpy as jnp
from jax import lax
from jax.experimental import pallas as pl
from jax.experimental.pallas import tpu as pltpu
from jax.experimental.pallas import tpu_sc as plsc

_VOCAB = 1000000
_COL = 64
_PAD = 128
_BATCH = 4096
_SEQ = 200

_N = _BATCH * _SEQ
_NW = 32
_PER_W = _N // _NW
_CHUNK = 128
_NCHUNK = _PER_W // _CHUNK
_NBUF = 4
_ITERS = _NCHUNK // _NBUF


def _gather_body(table_hbm, idx_hbm, out_hbm, idx_v, *scratch):
    bufs = scratch[:_NBUF]
    gsems = scratch[_NBUF:2 * _NBUF]
    wsems = scratch[2 * _NBUF:]

    wid = lax.axis_index("s") * 2 + lax.axis_index("c")
    base = wid * _PER_W
    pltpu.sync_copy(idx_hbm.at[pl.ds(base, _PER_W)], idx_v)

    def gstart(j, b):
        pltpu.async_copy(
            table_hbm.at[idx_v.at[pl.ds(j * _CHUNK, _CHUNK)]], bufs[b], gsems[b]
        )

    def gwait(j, b):
        pltpu.make_async_copy(
            table_hbm.at[idx_v.at[pl.ds(j * _CHUNK, _CHUNK)]], bufs[b], gsems[b]
        ).wait()

    def wstart(j, b):
        pltpu.async_copy(
            bufs[b],
            out_hbm.at[pl.ds(base + j * _CHUNK, _CHUNK), pl.ds(0, _COL)],
            wsems[b],
        )

    def wwait(j, b):
        pltpu.make_async_copy(
            bufs[b],
            out_hbm.at[pl.ds(base + j * _CHUNK, _CHUNK), pl.ds(0, _COL)],
            wsems[b],
        ).wait()

    for b in range(_NBUF):
        gstart(b, b)

    def body(it, _):
        for b in range(_NBUF):
            j = it * _NBUF + b
            gwait(j, b)
            wstart(j, b)

            @pl.when(it + 1 < _ITERS)
            def _():
                wwait(j, b)
                gstart(j + _NBUF, b)

        return 0

    lax.fori_loop(0, _ITERS, body, 0)

    for b in range(_NBUF):
        wwait((_ITERS - 1) * _NBUF + b, b)


def kernel(x, embed_weight):
    idx = x.reshape(_N).astype(jnp.int32)
    mesh = plsc.VectorSubcoreMesh(core_axis_name="c", subcore_axis_name="s")

    gather = functools.partial(
        pl.kernel,
        mesh=mesh,
        out_type=jax.ShapeDtypeStruct((_N, _PAD), jnp.float32),
        scratch_types=(
            [pltpu.VMEM((_PER_W,), jnp.int32)]
            + [pltpu.VMEM((_CHUNK, _COL), jnp.float32) for _ in range(_NBUF)]
            + [pltpu.SemaphoreType.DMA for _ in range(2 * _NBUF)]
        ),
        compiler_params=pltpu.CompilerParams(use_tc_tiling_on_sc=False),
    )(_gather_body)

    out = gather(embed_weight, idx)
    return out[:, :_COL].reshape(_BATCH, _SEQ, _COL)
